# Initial kernel scaffold; baseline (speedup 1.0000x reference)
import jax
import jax.numpy as jnp
from jax.experimental import pallas as pl


def kernel(xyz, params):
    raise NotImplementedError("write your pallas kernel here")


# speedup vs baseline: 4.8387x; 4.8387x over previous
import functools

import jax
import jax.numpy as jnp
from jax import lax
from jax.experimental import pallas as pl
from jax.experimental.pallas import tpu as pltpu
from jax.experimental.pallas import tpu_sc as plsc

_DIV = 4
_POS_HID = 64
_ATTN_MULT = 4
_DOWN_NP = [256, 64, 32, 16]
_DOWN_NS = [10, 16, 16, 16]
_UP_NS = [16, 16, 16, 10]
_PRE_NS = 16

_NC = 2
_NS = 16
_NW = _NC * _NS


@functools.lru_cache(maxsize=None)
def _sc_gather_fn(v_rows, d, bp):
    del v_rows
    b_per_w = bp // _NW
    chunk = 8
    for c in range(128, 7, -8):
        if b_per_w % c == 0:
            chunk = c
            break
    nch = b_per_w // chunk
    mesh = plsc.VectorSubcoreMesh(core_axis_name="c", subcore_axis_name="s")

    @functools.partial(
        pl.kernel,
        mesh=mesh,
        out_type=jax.ShapeDtypeStruct((bp, d), jnp.float32),
        scratch_types=[
            pltpu.VMEM((chunk,), jnp.int32),
            pltpu.VMEM((chunk, d), jnp.float32),
            pltpu.SemaphoreType.DMA,
        ],
    )
    def gather_kernel(table_hbm, idx_hbm, out_hbm, idx_v, rows_v, sem):
        wid = lax.axis_index("s") * _NC + lax.axis_index("c")

        def body(i, carry):
            base = wid * b_per_w + i * chunk
            pltpu.sync_copy(idx_hbm.at[pl.ds(base, chunk)], idx_v)
            pltpu.async_copy(table_hbm.at[idx_v], rows_v, sem).wait()
            pltpu.sync_copy(rows_v, out_hbm.at[pl.ds(base, chunk)])
            return carry

        lax.fori_loop(0, nch, body, 0)

    return gather_kernel


def _sc_gather(table, idx):
    v, d = table.shape
    (bp,) = idx.shape
    return _sc_gather_fn(v, d, bp)(table, idx)


def _pad_cols(x, width):
    rows, c = x.shape
    if c == width:
        return x
    return jnp.concatenate(
        [x, jnp.zeros((rows, width - c), jnp.float32)], axis=1)


def _round128(c):
    return ((c + 127) // 128) * 128


def _pad_idx(idx_flat):
    n = idx_flat.shape[0]
    npad = (-n) % 256
    if npad:
        idx_flat = jnp.concatenate(
            [idx_flat, jnp.zeros((npad,), jnp.int32)], axis=0)
    return idx_flat, n


def _gather_rows(table, idx_flat):
    pidx, n = _pad_idx(idx_flat.astype(jnp.int32))
    out = _sc_gather(table, pidx)
    return out[:n]


@functools.lru_cache(maxsize=None)
def _linear_fn(m, cin, co, affine, has_acc, act):
    mt = min(m, 2048)
    grid = (m // mt,)

    def body(*refs):
        x_ref, w_ref, b_ref = refs[:3]
        i = 3
        if affine:
            g_ref, beta_ref = refs[i:i + 2]
            i += 2
        if has_acc:
            acc_ref = refs[i]
            i += 1
        out_ref = refs[i]
        y = jnp.dot(x_ref[...], w_ref[...],
                    preferred_element_type=jnp.float32) + b_ref[...]
        if affine:
            y = y * g_ref[...] + beta_ref[...]
        if has_acc:
            y = y + acc_ref[...]
        if act == "relu":
            y = jnp.maximum(y, 0.0)
        elif act == "sigmoid":
            y = jax.nn.sigmoid(y)
        elif act == "softmax":
            y = y - jnp.max(y, axis=-1, keepdims=True)
            y = jnp.exp(y)
            y = y / jnp.sum(y, axis=-1, keepdims=True)
        out_ref[...] = y

    in_specs = [
        pl.BlockSpec((mt, cin), lambda i: (i, 0)),
        pl.BlockSpec((cin, co), lambda i: (0, 0)),
        pl.BlockSpec((1, co), lambda i: (0, 0)),
    ]
    if affine:
        in_specs += [pl.BlockSpec((1, co), lambda i: (0, 0))] * 2
    if has_acc:
        in_specs += [pl.BlockSpec((mt, co), lambda i: (i, 0))]

    return pl.pallas_call(
        body,
        grid=grid,
        in_specs=in_specs,
        out_specs=pl.BlockSpec((mt, co), lambda i: (i, 0)),
        out_shape=jax.ShapeDtypeStruct((m, co), jnp.float32),
    )


def _linear(x, w, b, g=None, beta=None, acc=None, act="none"):
    m, cin = x.shape
    co = w.shape[1]
    args = [x, w, b.reshape(1, co)]
    affine = g is not None
    if affine:
        args += [g.reshape(1, co), beta.reshape(1, co)]
    has_acc = acc is not None
    if has_acc:
        args += [acc]
    return _linear_fn(m, cin, co, affine, has_acc, act)(*args)


def _lin_bn_relu(x, p):
    return _linear(x, p['w'], p['b'], p['g'], p['beta'], act="relu")


@functools.lru_cache(maxsize=None)
def _knn_fn(b, nq, nk, k):
    qt = min(nq, 512)
    grid = (b, nq // qt)

    def body(pq_ref, pk_ref, out_ref):
        pq = pq_ref[0]
        d = jnp.zeros((qt, nk), jnp.float32)
        for c in range(3):
            qc = pq[:, c:c + 1]
            kc = pk_ref[0, c:c + 1, :]
            d = d + (qc - kc) ** 2
        iota = lax.broadcasted_iota(jnp.int32, (qt, nk), 1)
        cols = []
        for _ in range(k):
            m = jnp.min(d, axis=1, keepdims=True)
            idxj = jnp.min(jnp.where(d == m, iota, nk), axis=1,
                           keepdims=True)
            cols.append(idxj)
            d = jnp.where(iota == idxj, jnp.inf, d)
        out_ref[0] = jnp.concatenate(cols, axis=1)

    return pl.pallas_call(
        body,
        grid=grid,
        in_specs=[
            pl.BlockSpec((1, qt, 3), lambda bi, t: (bi, t, 0)),
            pl.BlockSpec((1, 3, nk), lambda bi, t: (bi, 0, 0)),
        ],
        out_specs=pl.BlockSpec((1, qt, k), lambda bi, t: (bi, t, 0)),
        out_shape=jax.ShapeDtypeStruct((b, nq, k), jnp.int32),
    )


def _knn(pq_nl, pk_ln, k):
    b, nq, _ = pq_nl.shape
    nk = pk_ln.shape[2]
    return _knn_fn(b, nq, nk, k)(pq_nl, pk_ln)


@functools.lru_cache(maxsize=None)
def _fps_fn(b, n, npoint):
    def body(pos_ref, out_ref):
        iota = lax.broadcasted_iota(jnp.int32, (1, n), 1)
        iota_np = lax.broadcasted_iota(jnp.int32, (1, npoint), 1)
        rows = []
        for bi in range(b):
            xr = pos_ref[bi, 0:1, :]
            yr = pos_ref[bi, 1:2, :]
            zr = pos_ref[bi, 2:3, :]

            def step(i, st):
                dist, prev, vec = st
                msk = iota == prev
                xs = jnp.sum(jnp.where(msk, xr, 0.0))
                ys = jnp.sum(jnp.where(msk, yr, 0.0))
                zs = jnp.sum(jnp.where(msk, zr, 0.0))
                dd = (xr - xs) ** 2 + (yr - ys) ** 2 + (zr - zs) ** 2
                dist = jnp.minimum(dist, dd)
                m = jnp.max(dist)
                nidx = jnp.min(jnp.where(dist == m, iota, n))
                vec = jnp.where(iota_np == i, nidx, vec)
                return dist, nidx, vec

            dist0 = jnp.full((1, n), 1e10, jnp.float32)
            vec0 = jnp.zeros((1, npoint), jnp.int32)
            _, _, vec = lax.fori_loop(1, npoint, step,
                                      (dist0, jnp.int32(0), vec0))
            rows.append(vec)
        out_ref[...] = jnp.concatenate(rows, axis=0)

    return pl.pallas_call(
        body,
        in_specs=[pl.BlockSpec((b, 3, n), lambda: (0, 0, 0))],
        out_specs=pl.BlockSpec((b, npoint), lambda: (0, 0)),
        out_shape=jax.ShapeDtypeStruct((b, npoint), jnp.int32),
    )


@functools.lru_cache(maxsize=None)
def _pt_attn_fn(b, n, k, c, d2):
    c4 = c // _DIV
    h = c * _ATTN_MULT
    nt = min(n, 128)
    grid = (b, n // nt)

    def body(q_ref, pos_ref, g_ref, pw1, pb1, pw2, pb2,
             aw1, ab1, aw2, ab2, out_ref):
        q = q_ref[0]
        pos_i = pos_ref[0]
        kjs, vjs, rels = [], [], []
        for j in range(k):
            gj = g_ref[0, j]
            kjs.append(gj[:, :c])
            vjs.append(gj[:, c:2 * c])
            rels.append(pos_i - gj[:, 2 * c:2 * c + 3])
        rel_cat = jnp.concatenate(rels, axis=0)
        pe_cat = jnp.dot(
            jnp.maximum(jnp.dot(rel_cat, pw1[...],
                                preferred_element_type=jnp.float32)
                        + pb1[...], 0.0),
            pw2[...], preferred_element_type=jnp.float32) + pb2[...]
        a_cat = jnp.concatenate([q - kj for kj in kjs], axis=0) + pe_cat
        hmid = jnp.maximum(jnp.dot(a_cat, aw1[...],
                                   preferred_element_type=jnp.float32)
                           + ab1[...], 0.0)
        logits = jnp.dot(hmid, aw2[...],
                         preferred_element_type=jnp.float32) + ab2[...]
        ls = [logits[j * nt:(j + 1) * nt] for j in range(k)]
        m = ls[0]
        for lj in ls[1:]:
            m = jnp.maximum(m, lj)
        es = [jnp.exp(lj - m) for lj in ls]
        s = es[0]
        for ej in es[1:]:
            s = s + ej
        out = jnp.zeros((nt, c), jnp.float32)
        for j in range(k):
            wexp = jnp.repeat(es[j] / s, _DIV, axis=1)
            pe_j = pe_cat[j * nt:(j + 1) * nt]
            out = out + wexp * (vjs[j] + pe_j)
        out_ref[0] = out

    def const(shape):
        return pl.BlockSpec(shape, lambda bi, t: tuple(0 for _ in shape))

    return pl.pallas_call(
        body,
        grid=grid,
        in_specs=[
            pl.BlockSpec((1, nt, c), lambda bi, t: (bi, t, 0)),
            pl.BlockSpec((1, nt, 3), lambda bi, t: (bi, t, 0)),
            pl.BlockSpec((1, k, nt, d2), lambda bi, t: (bi, 0, t, 0)),
            const((3, _POS_HID)), const((1, _POS_HID)),
            const((_POS_HID, c)), const((1, c)),
            const((c, h)), const((1, h)),
            const((h, c4)), const((1, c4)),
        ],
        out_specs=pl.BlockSpec((1, nt, c), lambda bi, t: (bi, t, 0)),
        out_shape=jax.ShapeDtypeStruct((b, n, c), jnp.float32),
    )


@functools.lru_cache(maxsize=None)
def _pool_fn(b, ns, npoint, co, dpad):
    def body(g_ref, out_ref):
        m = g_ref[0, 0][:, :co]
        for j in range(1, ns):
            m = jnp.maximum(m, g_ref[0, j][:, :co])
        out_ref[0] = m

    return pl.pallas_call(
        body,
        grid=(b,),
        in_specs=[pl.BlockSpec((1, ns, npoint, dpad),
                               lambda bi: (bi, 0, 0, 0))],
        out_specs=pl.BlockSpec((1, npoint, co), lambda bi: (bi, 0, 0)),
        out_shape=jax.ShapeDtypeStruct((b, npoint, co), jnp.float32),
    )


@functools.lru_cache(maxsize=None)
def _interp_fn(b, nf, cin, co, d2):
    nt = min(nf, 512)
    grid = (b, nf // nt)

    def body(g_ref, pos_ref, skip_ref, w1, b1, g1, beta1,
             w2, b2, g2, beta2, out_ref):
        pos_f = pos_ref[0]
        us, ws = [], []
        for j in range(3):
            gj = g_ref[0, j]
            us.append(gj[:, :cin])
            pj = gj[:, cin:cin + 3]
            d = jnp.zeros((nt, 1), jnp.float32)
            for cc in range(3):
                d = d + (pos_f[:, cc:cc + 1] - pj[:, cc:cc + 1]) ** 2
            ws.append(1.0 / (d + 1e-8))
        stot = (ws[0] + ws[1]) + ws[2]
        interp = jnp.zeros((nt, cin), jnp.float32)
        for j in range(3):
            interp = interp + us[j] * (ws[j] / stot)
        y1 = jnp.dot(interp, w1[...],
                     preferred_element_type=jnp.float32) + b1[...]
        y1 = jnp.maximum(y1 * g1[...] + beta1[...], 0.0)
        y2 = jnp.dot(skip_ref[0], w2[...],
                     preferred_element_type=jnp.float32) + b2[...]
        y2 = jnp.maximum(y2 * g2[...] + beta2[...], 0.0)
        out_ref[0] = y1 + y2

    def const(shape):
        return pl.BlockSpec(shape, lambda bi, t: tuple(0 for _ in shape))

    return pl.pallas_call(
        body,
        grid=grid,
        in_specs=[
            pl.BlockSpec((1, 3, nt, d2), lambda bi, t: (bi, 0, t, 0)),
            pl.BlockSpec((1, nt, 3), lambda bi, t: (bi, t, 0)),
            pl.BlockSpec((1, nt, co), lambda bi, t: (bi, t, 0)),
            const((cin, co)), const((1, co)), const((1, co)), const((1, co)),
            const((co, co)), const((1, co)), const((1, co)), const((1, co)),
        ],
        out_specs=pl.BlockSpec((1, nt, co), lambda bi, t: (bi, t, 0)),
        out_shape=jax.ShapeDtypeStruct((b, nf, co), jnp.float32),
    )


def _row_offsets(b, n, shape):
    off = jnp.arange(b, dtype=jnp.int32) * n
    return off.reshape((b,) + (1,) * (len(shape) - 1))


def _pt_layer(pos_nl, pos_ln, x_flat, p, k):
    b, n, _ = pos_nl.shape
    c = x_flat.shape[1]
    qkv_w = jnp.concatenate([p['wq'], p['wk'], p['wv']], axis=1)
    qkv = _linear(x_flat, qkv_w, jnp.zeros((3 * c,), jnp.float32))
    q = qkv[:, :c]
    kf = qkv[:, c:2 * c]
    v = qkv[:, 2 * c:]
    idx = _knn(pos_nl, pos_ln, k)
    gi = idx + _row_offsets(b, n, idx.shape)
    gi = jnp.transpose(gi, (0, 2, 1)).reshape(-1)
    pos_flat = pos_nl.reshape(b * n, 3)
    d2 = _round128(2 * c + 3)
    table = _pad_cols(
        jnp.concatenate([kf, v, pos_flat], axis=1), d2)
    g = _gather_rows(table, gi).reshape(b, k, n, d2)
    attn = _pt_attn_fn(b, n, k, c, d2)(
        q.reshape(b, n, c), pos_nl, g,
        p['pw1'], p['pb1'].reshape(1, -1), p['pw2'], p['pb2'].reshape(1, -1),
        p['aw1'], p['ab1'].reshape(1, -1), p['aw2'], p['ab2'].reshape(1, -1))
    return attn.reshape(b * n, c)


def _res_block(pos_nl, pos_ln, x_flat, p, k):
    t = _linear(x_flat, p['li_w'], p['li_b'])
    t = _pt_layer(pos_nl, pos_ln, t, p['attn'], k)
    return _linear(t, p['lo_w'], p['lo_b'], acc=x_flat)


def _pos_table(pos_nl):
    b, n, _ = pos_nl.shape
    return _pad_cols(pos_nl.reshape(b * n, 3), 128)


def _down_block(pos_nl, pos_ln, x_flat, p, npoint, nsample):
    b, n, _ = pos_nl.shape
    fi = _fps_fn(b, n, npoint)(pos_ln)
    gi = (fi + _row_offsets(b, n, fi.shape)).reshape(-1)
    new_pos = _gather_rows(_pos_table(pos_nl), gi)[:, :3]
    new_pos_nl = new_pos.reshape(b, npoint, 3)
    new_pos_ln = jnp.transpose(new_pos_nl, (0, 2, 1))
    y = _lin_bn_relu(x_flat, p['down'])
    co = y.shape[1]
    gidx = _knn(new_pos_nl, pos_ln, nsample)
    gg = gidx + _row_offsets(b, n, gidx.shape)
    gg = jnp.transpose(gg, (0, 2, 1)).reshape(-1)
    dpad = _round128(co)
    g = _gather_rows(_pad_cols(y, dpad), gg).reshape(b, nsample, npoint, dpad)
    y = _pool_fn(b, nsample, npoint, co, dpad)(g).reshape(b * npoint, co)
    for rp in p['res']:
        y = _res_block(new_pos_nl, new_pos_ln, y, rp, nsample)
    return new_pos_nl, new_pos_ln, y


def _up_block(pos_c_nl, pos_c_ln, pos_f_nl, x_c_flat, skip_flat, p, nsample):
    b, nc, _ = pos_c_nl.shape
    nf = pos_f_nl.shape[1]
    cin = x_c_flat.shape[1]
    co = skip_flat.shape[1]
    idx = _knn(pos_f_nl, pos_c_ln, 3)
    gi = idx + _row_offsets(b, nc, idx.shape)
    gi = jnp.transpose(gi, (0, 2, 1)).reshape(-1)
    d2 = _round128(cin + 3)
    table = _pad_cols(
        jnp.concatenate([x_c_flat, pos_c_nl.reshape(b * nc, 3)], axis=1), d2)
    g = _gather_rows(table, gi).reshape(b, 3, nf, d2)
    q1 = p['lin1']
    q2 = p['lin2']
    y = _interp_fn(b, nf, cin, co, d2)(
        g, pos_f_nl, skip_flat.reshape(b, nf, co),
        q1['w'], q1['b'].reshape(1, -1), q1['g'].reshape(1, -1),
        q1['beta'].reshape(1, -1),
        q2['w'], q2['b'].reshape(1, -1), q2['g'].reshape(1, -1),
        q2['beta'].reshape(1, -1)).reshape(b * nf, co)
    pos_f_ln = jnp.transpose(pos_f_nl, (0, 2, 1))
    for rp in p['res']:
        y = _res_block(pos_f_nl, pos_f_ln, y, rp, nsample)
    return y


def kernel(xyz, params):
    b = xyz.shape[0]
    n = xyz.shape[2]
    pos_ln = xyz
    pos_nl = jnp.transpose(xyz, (0, 2, 1))
    x = pos_nl.reshape(b * n, 3)
    for lp in params['pre0']:
        x = _lin_bn_relu(x, lp)
    x = _res_block(pos_nl, pos_ln, x, params['pre1'], _PRE_NS)
    pos_list = [(pos_nl, pos_ln)]
    feat_list = [x]
    for i, dp in enumerate(params['down']):
        pos_nl, pos_ln, x = _down_block(pos_nl, pos_ln, x, dp,
                                        _DOWN_NP[i], _DOWN_NS[i])
        pos_list.append((pos_nl, pos_ln))
        feat_list.append(x)
    for i, up in enumerate(params['up']):
        pc_nl, pc_ln = pos_list[-(i + 1)]
        pf_nl, _ = pos_list[-(i + 2)]
        x = _up_block(pc_nl, pc_ln, pf_nl, x, feat_list[-(i + 2)], up,
                      _UP_NS[i])
    outs = []
    for name in ['R', 'T', 'N', 'M']:
        hp = params['heads'][name]
        y = _lin_bn_relu(x, hp['l1'])
        act = {'R': 'none', 'T': 'none', 'N': 'sigmoid', 'M': 'softmax'}[name]
        y = _linear(y, hp['w2'], hp['b2'], act=act)
        od = y.shape[1]
        outs.append(jnp.transpose(y.reshape(b, n, od), (0, 2, 1)))
    return tuple(outs)
